# baseline (speedup 1.0000x reference)
import functools

import jax
import jax.numpy as jnp
from jax import lax
from jax.experimental import pallas as pl
from jax.experimental.pallas import tpu as pltpu
from jax.experimental.pallas import tpu_sc as plsc

_N_TIMES = 288
_B = 16384
_D_LINK = 32
_D_TIME = 8

_NC = 2
_NS = 16
_NW = _NC * _NS
_CHUNK = _B // _NW
_L = 16

_sc_mesh = plsc.VectorSubcoreMesh(core_axis_name="c", subcore_axis_name="s")


@functools.partial(
    pl.kernel,
    mesh=_sc_mesh,
    compiler_params=pltpu.CompilerParams(
        use_tc_tiling_on_sc=True, needs_layout_passes=False),
    out_type=[
        jax.ShapeDtypeStruct((_B, _D_LINK), jnp.float32),
        jax.ShapeDtypeStruct((_D_TIME, _B), jnp.float32),
    ],
    scratch_types=[
        pltpu.VMEM((_CHUNK,), jnp.int32),
        pltpu.VMEM((_CHUNK,), jnp.int32),
        pltpu.VMEM((_CHUNK, _D_LINK), jnp.float32),
        pltpu.VMEM((_N_TIMES * _D_TIME,), jnp.float32),
        pltpu.VMEM((_D_TIME, _CHUNK), jnp.float32),
        pltpu.SemaphoreType.DMA,
    ],
)
def _sc_links(link_idx_hbm, time_idx_hbm, link_tab_hbm, time_tab_hbm,
              le_out, te_out, li_v, ti_v, le_v, tt_v, te_v, sem_l):
    wid = lax.axis_index("s") * _NC + lax.axis_index("c")
    base = wid * _CHUNK
    pltpu.sync_copy(link_idx_hbm.at[pl.ds(base, _CHUNK)], li_v)
    pltpu.sync_copy(time_idx_hbm.at[pl.ds(base, _CHUNK)], ti_v)
    pltpu.sync_copy(time_tab_hbm, tt_v)

    def _fire(c, carry):
        lv = li_v[pl.ds(c * _L, _L)]
        for j in range(_L):
            pltpu.async_copy(
                link_tab_hbm.at[pl.ds(lv[j], 1), :],
                le_v.at[pl.ds(c * _L + j, 1), :], sem_l)
        return carry

    lax.fori_loop(0, _CHUNK // _L, _fire, 0)

    def _te(k, carry):
        tirow = ti_v[pl.ds(k * _L, _L)] * _D_TIME
        for c in range(_D_TIME):
            te_v[c, pl.ds(k * _L, _L)] = plsc.load_gather(tt_v, [tirow + c])
        return carry

    lax.fori_loop(0, _CHUNK // _L, _te, 0)

    def _drain(c, carry):
        pltpu.make_async_copy(
            link_tab_hbm.at[pl.ds(0, 1), :],
            le_v.at[pl.ds(0, 1), :], sem_l).wait()
        return carry

    lax.fori_loop(0, _CHUNK, _drain, 0)
    pltpu.sync_copy(le_v, le_out.at[pl.ds(base, _CHUNK)])
    pltpu.sync_copy(te_v, te_out.at[:, pl.ds(base, _CHUNK)])


@functools.partial(
    pl.kernel,
    mesh=_sc_mesh,
    compiler_params=pltpu.CompilerParams(
        use_tc_tiling_on_sc=True, needs_layout_passes=False),
    out_type=jax.ShapeDtypeStruct((_B,), jnp.float32),
    scratch_types=[
        pltpu.VMEM((_CHUNK,), jnp.int32),
        pltpu.VMEM((_CHUNK,), jnp.int32),
        pltpu.VMEM((4, 128), jnp.int32),
        pltpu.VMEM((_CHUNK,), jnp.float32),
        pltpu.VMEM((_CHUNK,), jnp.float32),
        pltpu.SemaphoreType.DMA,
    ],
)
def _sc_cross_add(link_idx_hbm, time_idx_hbm, cross_flat_hbm, y0_hbm,
                  y_out, li_v, ti_v, ci4_v, cr_v, y0_v, sem_c):
    wid = lax.axis_index("s") * _NC + lax.axis_index("c")
    base = wid * _CHUNK
    pltpu.sync_copy(link_idx_hbm.at[pl.ds(base, _CHUNK)], li_v)
    pltpu.sync_copy(time_idx_hbm.at[pl.ds(base, _CHUNK)], ti_v)

    for k in range(_CHUNK // _L):
        a = li_v[pl.ds(k * _L, _L)]
        b = ti_v[pl.ds(k * _L, _L)]
        g = k * _L
        ci4_v[g // 128, pl.ds(g % 128, _L)] = a * _N_TIMES + b

    cross_copies = [
        pltpu.async_copy(cross_flat_hbm.at[ci4_v.at[r]],
                         cr_v.at[pl.ds(r * 128, 128)], sem_c)
        for r in range(4)
    ]
    pltpu.sync_copy(y0_hbm.at[pl.ds(base, _CHUNK)], y0_v)
    for c in cross_copies:
        c.wait()
    for k in range(_CHUNK // _L):
        cr_v[pl.ds(k * _L, _L)] = (cr_v[pl.ds(k * _L, _L)]
                                   + y0_v[pl.ds(k * _L, _L)])
    pltpu.sync_copy(cr_v, y_out.at[pl.ds(base, _CHUNK)])


_GRID = 4
_BLK = _B // _GRID


def _mlp_body(le_ref, te_ref, w1aT_ref, w1bT_ref, b1_ref,
              w2T_ref, b2_ref, w3T_ref, b3_ref, out_ref):
    dot = functools.partial(
        lax.dot_general,
        dimension_numbers=(((1,), (0,)), ((), ())),
        precision=lax.Precision.DEFAULT,
    )
    leT = lax.transpose(le_ref[...], (1, 0))
    h = dot(w1aT_ref[...], leT) + dot(w1bT_ref[...], te_ref[...])
    h = jnp.maximum(h + b1_ref[...], 0.0)
    h = jnp.maximum(dot(w2T_ref[...], h) + b2_ref[...], 0.0)
    y = dot(w3T_ref[...], h)
    out_ref[...] = (y + b3_ref[...])[None]


@jax.jit
def _tc_mlp(le, te128, w1aT, w1bT, b1c, w2T, b2c, w3T, b3c):
    full = lambda i: (0, 0)
    return pl.pallas_call(
        _mlp_body,
        grid=(_GRID,),
        in_specs=[
            pl.BlockSpec((_BLK, _D_LINK), lambda i: (i, 0)),
            pl.BlockSpec((_D_TIME, _BLK), lambda i: (0, i)),
            pl.BlockSpec((128, _D_LINK), full),
            pl.BlockSpec((128, _D_TIME), full),
            pl.BlockSpec((128, 1), full),
            pl.BlockSpec((64, 128), full),
            pl.BlockSpec((64, 1), full),
            pl.BlockSpec((1, 64), full),
            pl.BlockSpec((1, 1), full),
        ],
        out_specs=pl.BlockSpec((1, 1, _BLK), lambda i: (i, 0, 0)),
        out_shape=jax.ShapeDtypeStruct((_GRID, 1, _BLK), jnp.float32),
    )(le, te128, w1aT, w1bT, b1c, w2T, b2c, w3T, b3c)


def kernel(link_idx, time_idx, link_table, time_table, cross_table,
           W1, b1, W2, b2, W3, b3):
    li = link_idx.astype(jnp.int32)
    ti = time_idx.astype(jnp.int32)
    le, teT = _sc_links(li, ti, link_table, time_table.reshape(-1))
    y0 = _tc_mlp(
        le, teT,
        W1[:_D_LINK].T, W1[_D_LINK:].T, b1[:, None],
        W2.T, b2[:, None], W3.reshape(1, 64), b3[None, :])
    return _sc_cross_add(li, ti, cross_table.reshape(-1), y0.reshape(_B))


# speedup vs baseline: 1.0600x; 1.0600x over previous
import functools

import jax
import jax.numpy as jnp
from jax import lax
from jax.experimental import pallas as pl
from jax.experimental.pallas import tpu as pltpu
from jax.experimental.pallas import tpu_sc as plsc

_N_TIMES = 288
_B = 16384
_D_LINK = 32
_D_TIME = 8

_NC = 2
_NS = 16
_NW = _NC * _NS
_CHUNK = _B // _NW
_L = 16

_sc_mesh = plsc.VectorSubcoreMesh(core_axis_name="c", subcore_axis_name="s")


@functools.partial(
    pl.kernel,
    mesh=_sc_mesh,
    compiler_params=pltpu.CompilerParams(
        use_tc_tiling_on_sc=True, needs_layout_passes=False),
    out_type=[
        jax.ShapeDtypeStruct((_B, _D_LINK), jnp.float32),
        jax.ShapeDtypeStruct((_D_TIME, _B), jnp.float32),
        jax.ShapeDtypeStruct((_B,), jnp.float32),
    ],
    scratch_types=[
        pltpu.VMEM((_CHUNK,), jnp.int32),
        pltpu.VMEM((_CHUNK,), jnp.int32),
        pltpu.VMEM((4, 128), jnp.int32),
        pltpu.VMEM((_CHUNK, _D_LINK), jnp.float32),
        pltpu.VMEM((_N_TIMES * _D_TIME,), jnp.float32),
        pltpu.VMEM((_D_TIME, _CHUNK), jnp.float32),
        pltpu.VMEM((_CHUNK,), jnp.float32),
        pltpu.SemaphoreType.DMA,
        pltpu.SemaphoreType.DMA,
    ],
)
def _sc_gather(link_idx_hbm, time_idx_hbm, link_tab_hbm, time_tab_hbm,
               cross_flat_hbm, le_out, te_out, cr_out,
               li_v, ti_v, ci4_v, le_v, tt_v, te_v, cr_v,
               sem_l, sem_c):
    wid = lax.axis_index("s") * _NC + lax.axis_index("c")
    base = wid * _CHUNK
    pltpu.sync_copy(link_idx_hbm.at[pl.ds(base, _CHUNK)], li_v)
    pltpu.sync_copy(time_idx_hbm.at[pl.ds(base, _CHUNK)], ti_v)
    pltpu.sync_copy(time_tab_hbm, tt_v)

    for k in range(_CHUNK // _L):
        a = li_v[pl.ds(k * _L, _L)]
        b = ti_v[pl.ds(k * _L, _L)]
        g = k * _L
        ci4_v[g // 128, pl.ds(g % 128, _L)] = a * _N_TIMES + b

    cross_copies = [
        pltpu.async_copy(cross_flat_hbm.at[ci4_v.at[r]],
                         cr_v.at[pl.ds(r * 128, 128)], sem_c)
        for r in range(4)
    ]

    def _fire(c, carry):
        lv = li_v[pl.ds(c * _L, _L)]
        for j in range(_L):
            pltpu.async_copy(
                link_tab_hbm.at[pl.ds(lv[j], 1), :],
                le_v.at[pl.ds(c * _L + j, 1), :], sem_l)
        return carry

    lax.fori_loop(0, _CHUNK // _L, _fire, 0)

    def _te(k, carry):
        tirow = ti_v[pl.ds(k * _L, _L)] * _D_TIME
        for c in range(_D_TIME):
            te_v[c, pl.ds(k * _L, _L)] = plsc.load_gather(tt_v, [tirow + c])
        return carry

    lax.fori_loop(0, _CHUNK // _L, _te, 0)

    def _drain(c, carry):
        pltpu.make_async_copy(
            link_tab_hbm.at[pl.ds(0, 1), :],
            le_v.at[pl.ds(0, 1), :], sem_l).wait()
        return carry

    lax.fori_loop(0, _CHUNK, _drain, 0)
    for c in cross_copies:
        c.wait()

    pltpu.sync_copy(le_v, le_out.at[pl.ds(base, _CHUNK)])
    pltpu.sync_copy(te_v, te_out.at[:, pl.ds(base, _CHUNK)])
    pltpu.sync_copy(cr_v, cr_out.at[pl.ds(base, _CHUNK)])


_GRID = 4
_BLK = _B // _GRID


def _mlp_body(le_ref, te_ref, cr_ref, w1aT_ref, w1bT_ref, b1_ref,
              w2T_ref, b2_ref, w3T_ref, b3_ref, out_ref):
    dot = functools.partial(
        lax.dot_general,
        dimension_numbers=(((1,), (0,)), ((), ())),
        precision=lax.Precision.DEFAULT,
    )
    leT = lax.transpose(le_ref[...], (1, 0))
    h = dot(w1aT_ref[...], leT) + dot(w1bT_ref[...], te_ref[...])
    h = jnp.maximum(h + b1_ref[...], 0.0)
    h = jnp.maximum(dot(w2T_ref[...], h) + b2_ref[...], 0.0)
    y = dot(w3T_ref[...], h)
    out_ref[...] = lax.reshape(y + b3_ref[...] + cr_ref[0], (_BLK,))


@jax.jit
def _tc_mlp(le, te128, cr3, w1aT, w1bT, b1c, w2T, b2c, w3T, b3c):
    full = lambda i: (0, 0)
    return pl.pallas_call(
        _mlp_body,
        grid=(_GRID,),
        in_specs=[
            pl.BlockSpec((_BLK, _D_LINK), lambda i: (i, 0)),
            pl.BlockSpec((_D_TIME, _BLK), lambda i: (0, i)),
            pl.BlockSpec((1, 1, _BLK), lambda i: (i, 0, 0)),
            pl.BlockSpec((128, _D_LINK), full),
            pl.BlockSpec((128, _D_TIME), full),
            pl.BlockSpec((128, 1), full),
            pl.BlockSpec((64, 128), full),
            pl.BlockSpec((64, 1), full),
            pl.BlockSpec((1, 64), full),
            pl.BlockSpec((1, 1), full),
        ],
        out_specs=pl.BlockSpec((_BLK,), lambda i: (i,)),
        out_shape=jax.ShapeDtypeStruct((_B,), jnp.float32),
    )(le, te128, cr3, w1aT, w1bT, b1c, w2T, b2c, w3T, b3c)


def kernel(link_idx, time_idx, link_table, time_table, cross_table,
           W1, b1, W2, b2, W3, b3):
    li = link_idx.astype(jnp.int32)
    ti = time_idx.astype(jnp.int32)
    le, teT, cr = _sc_gather(li, ti, link_table,
                             time_table.reshape(-1),
                             cross_table.reshape(-1))
    y = _tc_mlp(
        le, teT,
        cr.reshape(_GRID, 1, _BLK),
        W1[:_D_LINK].T, W1[_D_LINK:].T, b1[:, None],
        W2.T, b2[:, None], W3.reshape(1, 64), b3[None, :])
    return y


# speedup vs baseline: 1.0986x; 1.0365x over previous
import functools

import jax
import jax.numpy as jnp
from jax import lax
from jax.experimental import pallas as pl
from jax.experimental.pallas import tpu as pltpu
from jax.experimental.pallas import tpu_sc as plsc

_N_TIMES = 288
_B = 16384
_D_LINK = 32
_D_TIME = 8

_NC = 2
_NS = 16
_NW = _NC * _NS
_CHUNK = _B // _NW
_L = 16

_sc_mesh = plsc.VectorSubcoreMesh(core_axis_name="c", subcore_axis_name="s")


@functools.partial(
    pl.kernel,
    mesh=_sc_mesh,
    compiler_params=pltpu.CompilerParams(
        use_tc_tiling_on_sc=True, needs_layout_passes=False),
    out_type=[
        jax.ShapeDtypeStruct((_B, _D_LINK), jnp.float32),
        jax.ShapeDtypeStruct((_D_TIME, _B), jnp.float32),
        jax.ShapeDtypeStruct((_B,), jnp.float32),
    ],
    scratch_types=[
        pltpu.VMEM((_CHUNK,), jnp.int32),
        pltpu.VMEM((_CHUNK,), jnp.int32),
        pltpu.VMEM((4, 128), jnp.int32),
        pltpu.VMEM((_CHUNK, _D_LINK), jnp.float32),
        pltpu.VMEM((_N_TIMES * _D_TIME,), jnp.float32),
        pltpu.VMEM((_D_TIME, _CHUNK), jnp.float32),
        pltpu.VMEM((_CHUNK,), jnp.float32),
        pltpu.SemaphoreType.DMA,
        pltpu.SemaphoreType.DMA,
    ],
)
def _sc_gather(link_idx_hbm, time_idx_hbm, link_tab_hbm, time_tab_hbm,
               cross_flat_hbm, le_out, te_out, cr_out,
               li_v, ti_v, ci4_v, le_v, tt_v, te_v, cr_v,
               sem_l, sem_c):
    wid = lax.axis_index("s") * _NC + lax.axis_index("c")
    base = wid * _CHUNK
    pltpu.sync_copy(link_idx_hbm.at[pl.ds(base, _CHUNK)], li_v)
    pltpu.sync_copy(time_idx_hbm.at[pl.ds(base, _CHUNK)], ti_v)
    pltpu.sync_copy(time_tab_hbm, tt_v)

    for k in range(_CHUNK // _L):
        a = li_v[pl.ds(k * _L, _L)]
        b = ti_v[pl.ds(k * _L, _L)]
        g = k * _L
        ci4_v[g // 128, pl.ds(g % 128, _L)] = a * _N_TIMES + b

    cross_copies = [
        pltpu.async_copy(cross_flat_hbm.at[ci4_v.at[r]],
                         cr_v.at[pl.ds(r * 128, 128)], sem_c)
        for r in range(4)
    ]

    def _fire(c, carry):
        lv = li_v[pl.ds(c * _L, _L)]
        for j in range(_L):
            pltpu.async_copy(
                link_tab_hbm.at[pl.ds(lv[j], 1), :],
                le_v.at[pl.ds(c * _L + j, 1), :], sem_l)
        return carry

    lax.fori_loop(0, _CHUNK // _L, _fire, 0)

    def _te(k, carry):
        tirow = ti_v[pl.ds(k * _L, _L)] * _D_TIME
        for c in range(_D_TIME):
            te_v[c, pl.ds(k * _L, _L)] = plsc.load_gather(tt_v, [tirow + c])
        return carry

    lax.fori_loop(0, _CHUNK // _L, _te, 0)

    pltpu.make_async_copy(
        link_tab_hbm.at[pl.ds(0, _CHUNK), :], le_v, sem_l).wait()
    for c in cross_copies:
        c.wait()

    pltpu.sync_copy(le_v, le_out.at[pl.ds(base, _CHUNK)])
    pltpu.sync_copy(te_v, te_out.at[:, pl.ds(base, _CHUNK)])
    pltpu.sync_copy(cr_v, cr_out.at[pl.ds(base, _CHUNK)])


_GRID = 4
_BLK = _B // _GRID


def _mlp_body(le_ref, te_ref, cr_ref, w1aT_ref, w1bT_ref, b1_ref,
              w2T_ref, b2_ref, w3T_ref, b3_ref, out_ref):
    dot = functools.partial(
        lax.dot_general,
        dimension_numbers=(((1,), (0,)), ((), ())),
        precision=lax.Precision.DEFAULT,
    )
    leT = lax.transpose(le_ref[...], (1, 0))
    h = dot(w1aT_ref[...], leT) + dot(w1bT_ref[...], te_ref[...])
    h = jnp.maximum(h + b1_ref[...], 0.0)
    h = jnp.maximum(dot(w2T_ref[...], h) + b2_ref[...], 0.0)
    y = dot(w3T_ref[...], h)
    out_ref[...] = lax.reshape(y + b3_ref[...] + cr_ref[0], (_BLK,))


@jax.jit
def _tc_mlp(le, te128, cr3, w1aT, w1bT, b1c, w2T, b2c, w3T, b3c):
    full = lambda i: (0, 0)
    return pl.pallas_call(
        _mlp_body,
        grid=(_GRID,),
        in_specs=[
            pl.BlockSpec((_BLK, _D_LINK), lambda i: (i, 0)),
            pl.BlockSpec((_D_TIME, _BLK), lambda i: (0, i)),
            pl.BlockSpec((1, 1, _BLK), lambda i: (i, 0, 0)),
            pl.BlockSpec((128, _D_LINK), full),
            pl.BlockSpec((128, _D_TIME), full),
            pl.BlockSpec((128, 1), full),
            pl.BlockSpec((64, 128), full),
            pl.BlockSpec((64, 1), full),
            pl.BlockSpec((1, 64), full),
            pl.BlockSpec((1, 1), full),
        ],
        out_specs=pl.BlockSpec((_BLK,), lambda i: (i,)),
        out_shape=jax.ShapeDtypeStruct((_B,), jnp.float32),
    )(le, te128, cr3, w1aT, w1bT, b1c, w2T, b2c, w3T, b3c)


def kernel(link_idx, time_idx, link_table, time_table, cross_table,
           W1, b1, W2, b2, W3, b3):
    li = link_idx.astype(jnp.int32)
    ti = time_idx.astype(jnp.int32)
    le, teT, cr = _sc_gather(li, ti, link_table,
                             time_table.reshape(-1),
                             cross_table.reshape(-1))
    y = _tc_mlp(
        le, teT,
        cr.reshape(_GRID, 1, _BLK),
        W1[:_D_LINK].T, W1[_D_LINK:].T, b1[:, None],
        W2.T, b2[:, None], W3.reshape(1, 64), b3[None, :])
    return y


# speedup vs baseline: 1.1090x; 1.0095x over previous
import functools

import jax
import jax.numpy as jnp
from jax import lax
from jax.experimental import pallas as pl
from jax.experimental.pallas import tpu as pltpu
from jax.experimental.pallas import tpu_sc as plsc

_N_TIMES = 288
_B = 16384
_D_LINK = 32
_D_TIME = 8

_NC = 2
_NS = 16
_NW = _NC * _NS
_CHUNK = _B // _NW
_L = 16

_sc_mesh = plsc.VectorSubcoreMesh(core_axis_name="c", subcore_axis_name="s")


@functools.partial(
    pl.kernel,
    mesh=_sc_mesh,
    compiler_params=pltpu.CompilerParams(
        use_tc_tiling_on_sc=True, needs_layout_passes=False),
    out_type=[
        jax.ShapeDtypeStruct((_B, _D_LINK), jnp.float32),
        jax.ShapeDtypeStruct((_D_TIME, _B), jnp.float32),
        jax.ShapeDtypeStruct((_B,), jnp.float32),
    ],
    scratch_types=[
        pltpu.VMEM((_CHUNK,), jnp.int32),
        pltpu.VMEM((_CHUNK,), jnp.int32),
        pltpu.VMEM((4, 128), jnp.int32),
        pltpu.VMEM((_CHUNK, _D_LINK), jnp.float32),
        pltpu.VMEM((_N_TIMES * _D_TIME,), jnp.float32),
        pltpu.VMEM((_D_TIME, _CHUNK), jnp.float32),
        pltpu.VMEM((_CHUNK,), jnp.float32),
        pltpu.SemaphoreType.DMA,
        pltpu.SemaphoreType.DMA,
    ],
)
def _sc_gather(link_idx_hbm, time_idx_hbm, link_tab_hbm, time_tab_hbm,
               cross_flat_hbm, le_out, te_out, cr_out,
               li_v, ti_v, ci4_v, le_v, tt_v, te_v, cr_v,
               sem_l, sem_c):
    wid = lax.axis_index("s") * _NC + lax.axis_index("c")
    base = wid * _CHUNK
    pltpu.sync_copy(link_idx_hbm.at[pl.ds(base, _CHUNK)], li_v)
    pltpu.sync_copy(time_idx_hbm.at[pl.ds(base, _CHUNK)], ti_v)
    pltpu.sync_copy(time_tab_hbm, tt_v)

    for k in range(_CHUNK // _L):
        a = li_v[pl.ds(k * _L, _L)]
        b = ti_v[pl.ds(k * _L, _L)]
        g = k * _L
        ci4_v[g // 128, pl.ds(g % 128, _L)] = a * _N_TIMES + b

    cross_copies = [
        pltpu.async_copy(cross_flat_hbm.at[ci4_v.at[r]],
                         cr_v.at[pl.ds(r * 128, 128)], sem_c)
        for r in range(4)
    ]

    def _fire(c, carry):
        lv = li_v[pl.ds(c * _L, _L)]
        for j in range(_L):
            pltpu.async_copy(
                link_tab_hbm.at[pl.ds(lv[j], 1), :],
                le_v.at[pl.ds(c * _L + j, 1), :], sem_l)
        return carry

    lax.fori_loop(0, _CHUNK // _L, _fire, 0)

    def _te(k, carry):
        tirow = ti_v[pl.ds(k * _L, _L)] * _D_TIME
        for c in range(_D_TIME):
            te_v[c, pl.ds(k * _L, _L)] = plsc.load_gather(tt_v, [tirow + c])
        return carry

    lax.fori_loop(0, _CHUNK // _L, _te, 0)

    pltpu.make_async_copy(
        link_tab_hbm.at[pl.ds(0, _CHUNK), :], le_v, sem_l).wait()
    for c in cross_copies:
        c.wait()

    pltpu.sync_copy(le_v, le_out.at[pl.ds(base, _CHUNK)])
    pltpu.sync_copy(te_v, te_out.at[:, pl.ds(base, _CHUNK)])
    pltpu.sync_copy(cr_v, cr_out.at[pl.ds(base, _CHUNK)])


_GRID = 2
_BLK = _B // _GRID


def _mlp_body(le_ref, te_ref, cr_ref, w1aT_ref, w1bT_ref, b1_ref,
              w2T_ref, b2_ref, w3T_ref, b3_ref, out_ref):
    dot = functools.partial(
        lax.dot_general,
        dimension_numbers=(((1,), (0,)), ((), ())),
        precision=lax.Precision.DEFAULT,
    )
    leT = lax.transpose(le_ref[...], (1, 0))
    h = dot(w1aT_ref[...], leT) + dot(w1bT_ref[...], te_ref[...])
    h = jnp.maximum(h + b1_ref[...], 0.0)
    h = jnp.maximum(dot(w2T_ref[...], h) + b2_ref[...], 0.0)
    y = dot(w3T_ref[...], h)
    out_ref[...] = lax.reshape(y + b3_ref[...] + cr_ref[0], (_BLK,))


@jax.jit
def _tc_mlp(le, te128, cr3, w1aT, w1bT, b1c, w2T, b2c, w3T, b3c):
    full = lambda i: (0, 0)
    return pl.pallas_call(
        _mlp_body,
        grid=(_GRID,),
        in_specs=[
            pl.BlockSpec((_BLK, _D_LINK), lambda i: (i, 0)),
            pl.BlockSpec((_D_TIME, _BLK), lambda i: (0, i)),
            pl.BlockSpec((1, 1, _BLK), lambda i: (i, 0, 0)),
            pl.BlockSpec((128, _D_LINK), full),
            pl.BlockSpec((128, _D_TIME), full),
            pl.BlockSpec((128, 1), full),
            pl.BlockSpec((64, 128), full),
            pl.BlockSpec((64, 1), full),
            pl.BlockSpec((1, 64), full),
            pl.BlockSpec((1, 1), full),
        ],
        out_specs=pl.BlockSpec((_BLK,), lambda i: (i,)),
        out_shape=jax.ShapeDtypeStruct((_B,), jnp.float32),
    )(le, te128, cr3, w1aT, w1bT, b1c, w2T, b2c, w3T, b3c)


def kernel(link_idx, time_idx, link_table, time_table, cross_table,
           W1, b1, W2, b2, W3, b3):
    li = link_idx.astype(jnp.int32)
    ti = time_idx.astype(jnp.int32)
    le, teT, cr = _sc_gather(li, ti, link_table,
                             time_table.reshape(-1),
                             cross_table.reshape(-1))
    y = _tc_mlp(
        le, teT,
        cr.reshape(_GRID, 1, _BLK),
        W1[:_D_LINK].T, W1[_D_LINK:].T, b1[:, None],
        W2.T, b2[:, None], W3.reshape(1, 64), b3[None, :])
    return y
